# Initial kernel scaffold; baseline (speedup 1.0000x reference)
import jax
import jax.numpy as jnp
from jax.experimental import pallas as pl


def kernel(xyz, points, w1, b1, w2, b2, w3, b3, lin_w, lin_b):
    raise NotImplementedError("write your pallas kernel here")


# speedup vs baseline: 14.7845x; 14.7845x over previous
import functools

import jax
import jax.numpy as jnp
from jax import lax
from jax.experimental import pallas as pl
from jax.experimental.pallas import tpu as pltpu

B = 2
N = 8192
D = 64
NPOINT = 1024
NSAMPLE = 16
WEIGHTNET = 16
IN_CHANNEL = 3 + D
OUT_CHANNEL = 64
SUBR = 8
LANES = N // SUBR


def _fps_body(xyz48_ref, xyzt2_ref, newxyz_ref):
    row = lax.broadcasted_iota(jnp.int32, (SUBR, LANES), 0)
    col = lax.broadcasted_iota(jnp.int32, (SUBR, LANES), 1)
    gidx = row * LANES + col
    big = jnp.int32(2 * N)
    zero2 = jnp.zeros((1, 2), dtype=jnp.float32)

    def body(i, carry):
        dist, f0, f1 = carry
        c0 = xyzt2_ref[pl.ds(f0, 1), :]
        c1 = xyzt2_ref[pl.ds(f1 + N, 1), :]
        colv = jnp.concatenate(
            [c0[0:1, 0:1], c1[0:1, 0:1],
             c0[0:1, 1:2], c1[0:1, 1:2],
             c0[0:1, 2:3], c1[0:1, 2:3]], axis=0)
        rowv = jnp.concatenate(
            [c0[0:1, 0:1], c1[0:1, 0:1],
             c0[0:1, 1:2], c1[0:1, 1:2],
             c0[0:1, 2:3], c1[0:1, 2:3], zero2], axis=1)
        newxyz_ref[pl.ds(i, 1), :] = rowv
        c48 = jnp.repeat(colv, SUBR, axis=0)
        diff = xyz48_ref[...] - c48
        sq = diff * diff
        d = (sq[0:16] + sq[16:32]) + sq[32:48]
        dist = jnp.minimum(dist, d)
        d0 = dist[0:SUBR]
        d1 = dist[SUBR:2 * SUBR]
        m0 = jnp.max(d0)
        m1 = jnp.max(d1)
        nf0 = jnp.min(jnp.where(d0 == m0, gidx, big)).astype(jnp.int32)
        nf1 = jnp.min(jnp.where(d1 == m1, gidx, big)).astype(jnp.int32)
        return dist, nf0, nf1

    dist0 = jnp.full((2 * SUBR, LANES), 1e10, dtype=jnp.float32)
    lax.fori_loop(0, NPOINT, body,
                  (dist0, jnp.int32(0), jnp.int32(0)))


def _run_fps(xyz):
    xyz48 = xyz.reshape(B, 3, SUBR, LANES).transpose(1, 0, 2, 3).reshape(
        3 * B * SUBR, LANES)
    xyzt2 = xyz.transpose(0, 2, 1).reshape(B * N, 3)
    newxyz8 = pl.pallas_call(
        _fps_body,
        out_shape=jax.ShapeDtypeStruct((NPOINT, 8), jnp.float32),
    )(xyz48, xyzt2)
    return newxyz8[:, :6].reshape(NPOINT, 3, B).transpose(2, 1, 0)


QBLK = 256


def _knn_body(q_ref, xyz3_ref, knn_ref):
    q = q_ref[0]
    xyz3 = xyz3_ref[0]
    mm = jnp.dot(q, xyz3, preferred_element_type=jnp.float32)
    qn = jnp.sum(q * q, axis=1, keepdims=True)
    xn = jnp.sum(xyz3 * xyz3, axis=0, keepdims=True)
    d = (-2.0 * mm + qn) + xn
    iota = lax.broadcasted_iota(jnp.int32, (QBLK, N), 1)
    big = jnp.int32(2 * N)
    cols = []
    for _ in range(NSAMPLE):
        m = jnp.min(d, axis=1, keepdims=True)
        sel = d == m
        idx = jnp.min(jnp.where(sel, iota, big), axis=1, keepdims=True)
        cols.append(idx)
        d = jnp.where(iota == idx, jnp.float32(jnp.inf), d)
    knn_ref[0] = jnp.concatenate(cols, axis=1).astype(jnp.int32)


def _run_knn(new_xyz, xyz):
    q = new_xyz.transpose(0, 2, 1)
    grid = (B, NPOINT // QBLK)
    return pl.pallas_call(
        _knn_body,
        grid=grid,
        in_specs=[
            pl.BlockSpec((1, QBLK, 3), lambda b, j: (b, j, 0)),
            pl.BlockSpec((1, 3, N), lambda b, j: (b, 0, 0)),
        ],
        out_specs=pl.BlockSpec((1, QBLK, NSAMPLE), lambda b, j: (b, j, 0)),
        out_shape=jax.ShapeDtypeStruct((B, NPOINT, NSAMPLE), jnp.int32),
    )(q, xyz)


GROW = 3 + 13 + D
NWORK = 32
RPW = (B * NPOINT * NSAMPLE) // NWORK
ICHUNK = 128


DQ = 128


def _dense_body(g_ref, nx_ref, w1t_ref, b1_ref, w2t_ref, b2_ref, w3t_ref,
                b3_ref, L_ref, T16_ref, S_ref, rep_ref, linb_ref, out_ref):
    g = g_ref[...]
    nx = nx_ref[...]
    rep = jnp.dot(rep_ref[...], nx, preferred_element_type=jnp.float32)
    rel = g[:, 0:3] - rep
    h = jnp.maximum(
        jnp.dot(rel, w1t_ref[...], preferred_element_type=jnp.float32)
        + b1_ref[...], 0.0)
    h = jnp.maximum(
        jnp.dot(h, w2t_ref[...], preferred_element_type=jnp.float32)
        + b2_ref[...], 0.0)
    w = jnp.maximum(
        jnp.dot(h, w3t_ref[...], preferred_element_type=jnp.float32)
        + b3_ref[...], 0.0)
    npmat = jnp.concatenate([rel, g[:, 16:GROW]], axis=1)
    C = jnp.dot(npmat, L_ref[...], preferred_element_type=jnp.float32)
    Wt = jnp.dot(w, T16_ref[...], preferred_element_type=jnp.float32)
    R = jnp.dot(C * Wt, S_ref[...], preferred_element_type=jnp.float32)
    o = jnp.sum(R.reshape(DQ, NSAMPLE, OUT_CHANNEL), axis=1)
    o = o + linb_ref[...]
    out_ref[...] = jnp.where(o > 0, o, 0.1 * o)


def _run_dense(gathered, new_xyz, w1, b1, w2, b2, w3, b3, lin_w, lin_b):
    nxq = new_xyz.transpose(0, 2, 1).reshape(B * NPOINT, 3)
    L = lin_w.reshape(OUT_CHANNEL, IN_CHANNEL, WEIGHTNET).transpose(
        1, 0, 2).reshape(IN_CHANNEL, OUT_CHANNEL * WEIGHTNET)
    T16 = jnp.tile(jnp.eye(WEIGHTNET, dtype=jnp.float32), (1, OUT_CHANNEL))
    S = jnp.repeat(jnp.eye(OUT_CHANNEL, dtype=jnp.float32), WEIGHTNET,
                   axis=0)
    rep = jnp.repeat(jnp.eye(DQ, dtype=jnp.float32), NSAMPLE, axis=0)
    nblk = (B * NPOINT) // DQ
    full = lambda shape: pl.BlockSpec(shape, lambda j: tuple(0 for _ in shape))
    out = pl.pallas_call(
        _dense_body,
        grid=(nblk,),
        in_specs=[
            pl.BlockSpec((DQ * NSAMPLE, GROW), lambda j: (j, 0)),
            pl.BlockSpec((DQ, 3), lambda j: (j, 0)),
            full((3, 8)), full((1, 8)), full((8, 8)), full((1, 8)),
            full((8, WEIGHTNET)), full((1, WEIGHTNET)),
            full((IN_CHANNEL, OUT_CHANNEL * WEIGHTNET)),
            full((WEIGHTNET, OUT_CHANNEL * WEIGHTNET)),
            full((OUT_CHANNEL * WEIGHTNET, OUT_CHANNEL)),
            full((DQ * NSAMPLE, DQ)),
            full((1, OUT_CHANNEL)),
        ],
        out_specs=pl.BlockSpec((DQ, OUT_CHANNEL), lambda j: (j, 0)),
        out_shape=jax.ShapeDtypeStruct((B * NPOINT, OUT_CHANNEL),
                                       jnp.float32),
    )(gathered, nxq, w1.T, b1.reshape(1, 8), w2.T, b2.reshape(1, 8),
      w3.T, b3.reshape(1, WEIGHTNET), L, T16, S, rep,
      lin_b.reshape(1, OUT_CHANNEL))
    return out


def kernel(xyz, points, w1, b1, w2, b2, w3, b3, lin_w, lin_b):
    new_xyz = _run_fps(xyz)
    knn = _run_knn(new_xyz, xyz)
    tbl = jnp.concatenate(
        [xyz.transpose(0, 2, 1),
         jnp.zeros((B, N, 13), dtype=jnp.float32),
         points.transpose(0, 2, 1)], axis=-1).reshape(B * N, GROW)
    gidx = (knn + (jnp.arange(B, dtype=jnp.int32) * N)[:, None, None]
            ).reshape(B * NPOINT * NSAMPLE)
    gathered = tbl[gidx]
    out = _run_dense(gathered, new_xyz, w1, b1, w2, b2, w3, b3, lin_w, lin_b)
    out = out.reshape(B, NPOINT, OUT_CHANNEL).transpose(0, 2, 1)
    return new_xyz, out


# speedup vs baseline: 15.7424x; 1.0648x over previous
import functools

import jax
import jax.numpy as jnp
from jax import lax
from jax.experimental import pallas as pl
from jax.experimental.pallas import tpu as pltpu
from jax.experimental.pallas import tpu_sc as plsc

B = 2
N = 8192
D = 64
NPOINT = 1024
NSAMPLE = 16
WEIGHTNET = 16
IN_CHANNEL = 3 + D
OUT_CHANNEL = 64
SUBR = 8
LANES = N // SUBR


def _fps_body(xyz48_ref, xyzt2_ref, newxyz_ref):
    row = lax.broadcasted_iota(jnp.int32, (SUBR, LANES), 0)
    col = lax.broadcasted_iota(jnp.int32, (SUBR, LANES), 1)
    gidx = row * LANES + col
    big = jnp.int32(2 * N)
    zero2 = jnp.zeros((1, 2), dtype=jnp.float32)

    def body(i, carry):
        dist, f0, f1 = carry
        c0 = xyzt2_ref[pl.ds(f0, 1), :]
        c1 = xyzt2_ref[pl.ds(f1 + N, 1), :]
        colv = jnp.concatenate(
            [c0[0:1, 0:1], c1[0:1, 0:1],
             c0[0:1, 1:2], c1[0:1, 1:2],
             c0[0:1, 2:3], c1[0:1, 2:3]], axis=0)
        rowv = jnp.concatenate(
            [c0[0:1, 0:1], c1[0:1, 0:1],
             c0[0:1, 1:2], c1[0:1, 1:2],
             c0[0:1, 2:3], c1[0:1, 2:3], zero2], axis=1)
        newxyz_ref[pl.ds(i, 1), :] = rowv
        c48 = jnp.repeat(colv, SUBR, axis=0)
        diff = xyz48_ref[...] - c48
        sq = diff * diff
        d = (sq[0:16] + sq[16:32]) + sq[32:48]
        dist = jnp.minimum(dist, d)
        d0 = dist[0:SUBR]
        d1 = dist[SUBR:2 * SUBR]
        m0 = jnp.max(d0)
        m1 = jnp.max(d1)
        nf0 = jnp.min(jnp.where(d0 == m0, gidx, big)).astype(jnp.int32)
        nf1 = jnp.min(jnp.where(d1 == m1, gidx, big)).astype(jnp.int32)
        return dist, nf0, nf1

    dist0 = jnp.full((2 * SUBR, LANES), 1e10, dtype=jnp.float32)
    lax.fori_loop(0, NPOINT, body,
                  (dist0, jnp.int32(0), jnp.int32(0)))


def _run_fps(xyz):
    xyz48 = xyz.reshape(B, 3, SUBR, LANES).transpose(1, 0, 2, 3).reshape(
        3 * B * SUBR, LANES)
    xyzt2 = xyz.transpose(0, 2, 1).reshape(B * N, 3)
    newxyz8 = pl.pallas_call(
        _fps_body,
        out_shape=jax.ShapeDtypeStruct((NPOINT, 8), jnp.float32),
    )(xyz48, xyzt2)
    return newxyz8[:, :6].reshape(NPOINT, 3, B).transpose(2, 1, 0)


QBLK = 256


def _knn_body(q_ref, xyz3_ref, knn_ref):
    q = q_ref[0]
    xyz3 = xyz3_ref[0]
    mm = jnp.dot(q, xyz3, preferred_element_type=jnp.float32)
    qn = jnp.sum(q * q, axis=1, keepdims=True)
    xn = jnp.sum(xyz3 * xyz3, axis=0, keepdims=True)
    d = (-2.0 * mm + qn) + xn
    iota = lax.broadcasted_iota(jnp.int32, (QBLK, N), 1)
    big = jnp.int32(2 * N)
    cols = []
    for _ in range(NSAMPLE):
        m = jnp.min(d, axis=1, keepdims=True)
        sel = d == m
        idx = jnp.min(jnp.where(sel, iota, big), axis=1, keepdims=True)
        cols.append(idx)
        d = jnp.where(iota == idx, jnp.float32(jnp.inf), d)
    knn_ref[0] = jnp.concatenate(cols, axis=1).astype(jnp.int32)


def _run_knn(new_xyz, xyz):
    q = new_xyz.transpose(0, 2, 1)
    grid = (B, NPOINT // QBLK)
    return pl.pallas_call(
        _knn_body,
        grid=grid,
        in_specs=[
            pl.BlockSpec((1, QBLK, 3), lambda b, j: (b, j, 0)),
            pl.BlockSpec((1, 3, N), lambda b, j: (b, 0, 0)),
        ],
        out_specs=pl.BlockSpec((1, QBLK, NSAMPLE), lambda b, j: (b, j, 0)),
        out_shape=jax.ShapeDtypeStruct((B, NPOINT, NSAMPLE), jnp.int32),
    )(q, xyz)


GROW = 128
NWORK = 32
RPW = (B * NPOINT * NSAMPLE) // NWORK
ICHUNK = 128
NCH = RPW // ICHUNK
HALF = RPW // 2


def _gather_sc_body(tbl_hbm, idx_hbm, out_hbm, idx_v, rows_v, sem):
    wid = lax.axis_index("s") * 2 + lax.axis_index("c")
    pltpu.sync_copy(idx_hbm.at[wid], idx_v)
    for h in range(2):
        cps = []
        for j in range(NCH // 2):
            cps.append(pltpu.async_copy(
                tbl_hbm.at[idx_v.at[h * (NCH // 2) + j]],
                rows_v.at[pl.ds(j * ICHUNK, ICHUNK)], sem))
        for cp in cps:
            cp.wait()
        pltpu.sync_copy(rows_v, out_hbm.at[pl.ds(wid * RPW + h * HALF, HALF)])


def _run_gather_sc(tbl, gidx):
    idx3 = gidx.reshape(NWORK, NCH, ICHUNK)
    mesh = plsc.VectorSubcoreMesh(core_axis_name="c", subcore_axis_name="s")
    f = functools.partial(
        pl.kernel, mesh=mesh,
        out_type=jax.ShapeDtypeStruct((B * NPOINT * NSAMPLE, GROW),
                                      jnp.float32),
        scratch_types=[
            pltpu.VMEM((NCH, ICHUNK), jnp.int32),
            pltpu.VMEM((HALF, GROW), jnp.float32),
            pltpu.SemaphoreType.DMA,
        ],
    )(_gather_sc_body)
    return f(tbl, idx3)


DQ = 128


def _dense_body(g_ref, nx_ref, w1t_ref, b1_ref, w2t_ref, b2_ref, w3t_ref,
                b3_ref, L_ref, T16_ref, S_ref, rep_ref, linb_ref, out_ref):
    g = g_ref[...]
    nx = nx_ref[...]
    rep = jnp.dot(rep_ref[...], nx, preferred_element_type=jnp.float32)
    rel = g[:, 0:3] - rep
    h = jnp.maximum(
        jnp.dot(rel, w1t_ref[...], preferred_element_type=jnp.float32)
        + b1_ref[...], 0.0)
    h = jnp.maximum(
        jnp.dot(h, w2t_ref[...], preferred_element_type=jnp.float32)
        + b2_ref[...], 0.0)
    w = jnp.maximum(
        jnp.dot(h, w3t_ref[...], preferred_element_type=jnp.float32)
        + b3_ref[...], 0.0)
    npmat = jnp.concatenate([rel, g[:, 16:80]], axis=1)
    C = jnp.dot(npmat, L_ref[...], preferred_element_type=jnp.float32)
    Wt = jnp.dot(w, T16_ref[...], preferred_element_type=jnp.float32)
    R = jnp.dot(C * Wt, S_ref[...], preferred_element_type=jnp.float32)
    o = jnp.sum(R.reshape(DQ, NSAMPLE, OUT_CHANNEL), axis=1)
    o = o + linb_ref[...]
    out_ref[...] = jnp.where(o > 0, o, 0.1 * o)


def _run_dense(gathered, new_xyz, w1, b1, w2, b2, w3, b3, lin_w, lin_b):
    nxq = new_xyz.transpose(0, 2, 1).reshape(B * NPOINT, 3)
    L = lin_w.reshape(OUT_CHANNEL, IN_CHANNEL, WEIGHTNET).transpose(
        1, 0, 2).reshape(IN_CHANNEL, OUT_CHANNEL * WEIGHTNET)
    T16 = jnp.tile(jnp.eye(WEIGHTNET, dtype=jnp.float32), (1, OUT_CHANNEL))
    S = jnp.repeat(jnp.eye(OUT_CHANNEL, dtype=jnp.float32), WEIGHTNET,
                   axis=0)
    rep = jnp.repeat(jnp.eye(DQ, dtype=jnp.float32), NSAMPLE, axis=0)
    nblk = (B * NPOINT) // DQ
    full = lambda shape: pl.BlockSpec(shape, lambda j: tuple(0 for _ in shape))
    out = pl.pallas_call(
        _dense_body,
        grid=(nblk,),
        in_specs=[
            pl.BlockSpec((DQ * NSAMPLE, GROW), lambda j: (j, 0)),
            pl.BlockSpec((DQ, 3), lambda j: (j, 0)),
            full((3, 8)), full((1, 8)), full((8, 8)), full((1, 8)),
            full((8, WEIGHTNET)), full((1, WEIGHTNET)),
            full((IN_CHANNEL, OUT_CHANNEL * WEIGHTNET)),
            full((WEIGHTNET, OUT_CHANNEL * WEIGHTNET)),
            full((OUT_CHANNEL * WEIGHTNET, OUT_CHANNEL)),
            full((DQ * NSAMPLE, DQ)),
            full((1, OUT_CHANNEL)),
        ],
        out_specs=pl.BlockSpec((DQ, OUT_CHANNEL), lambda j: (j, 0)),
        out_shape=jax.ShapeDtypeStruct((B * NPOINT, OUT_CHANNEL),
                                       jnp.float32),
    )(gathered, nxq, w1.T, b1.reshape(1, 8), w2.T, b2.reshape(1, 8),
      w3.T, b3.reshape(1, WEIGHTNET), L, T16, S, rep,
      lin_b.reshape(1, OUT_CHANNEL))
    return out


def kernel(xyz, points, w1, b1, w2, b2, w3, b3, lin_w, lin_b):
    new_xyz = _run_fps(xyz)
    knn = _run_knn(new_xyz, xyz)
    tbl = jnp.concatenate(
        [xyz.transpose(0, 2, 1),
         jnp.zeros((B, N, 13), dtype=jnp.float32),
         points.transpose(0, 2, 1),
         jnp.zeros((B, N, GROW - 80), dtype=jnp.float32)],
        axis=-1).reshape(B * N, GROW)
    gidx = (knn + (jnp.arange(B, dtype=jnp.int32) * N)[:, None, None]
            ).reshape(B * NPOINT * NSAMPLE)
    gathered = _run_gather_sc(tbl, gidx)
    out = _run_dense(gathered, new_xyz, w1, b1, w2, b2, w3, b3, lin_w, lin_b)
    out = out.reshape(B, NPOINT, OUT_CHANNEL).transpose(0, 2, 1)
    return new_xyz, out
